# baseline (speedup 1.0000x reference)
import jax
import jax.numpy as jnp
from jax import lax
from jax.experimental import pallas as pl
from jax.experimental.pallas import tpu as pltpu

B = 4096
TB = 256
H = 128
HI = lax.Precision.HIGHEST


def _mm(a_bf, b_bf):
    return jnp.dot(a_bf, b_bf, precision=lax.Precision.DEFAULT,
                   preferred_element_type=jnp.float32)


def _bf(x):
    return x.astype(jnp.bfloat16)


def _quant(zres, cb_t_bf, cb):
    rs = jnp.sum(zres * zres, axis=1, keepdims=True)
    cs = jnp.sum(cb * cb, axis=1)[None, :]
    d = rs + cs - 2.0 * _mm(_bf(zres), cb_t_bf)
    m = jnp.min(d, axis=1, keepdims=True)
    ii = lax.broadcasted_iota(jnp.int32, d.shape, 1)
    idx = jnp.min(jnp.where(d == m, ii, d.shape[1]), axis=1, keepdims=True)
    oh = (ii == idx).astype(jnp.float32)
    zq = jnp.dot(oh, cb, precision=HI, preferred_element_type=jnp.float32)
    return zq, idx


def _fused_kernel(xs_ref, xc_ref,
                  sew0, seb0, sew1, seb1, sew2, seb2,
                  cew0, ceb0, cew1, ceb1, cew2, ceb2,
                  sdw0, sdb0, sdw1, sdb1, sdw2, sdb2,
                  cdw0, cdb0, cdw1, cdb1, cdw2, cdb2,
                  cbs_t, cbm0_t, cbm1_t, cbc0_t, cbc1_t,
                  cbs, cbm0, cbm1, cbc0, cbc1,
                  semo_ref, colo_ref, idx_ref, loss_ref):
    h = jnp.maximum(_mm(_bf(xs_ref[...]), sew0[...]) + seb0[...], 0.0)
    h = jnp.maximum(_mm(_bf(h), sew1[...]) + seb1[...], 0.0)
    zs = _mm(_bf(h), sew2[...]) + seb2[...]
    h = jnp.maximum(_mm(_bf(xc_ref[...]), cew0[...]) + ceb0[...], 0.0)
    h = jnp.maximum(_mm(_bf(h), cew1[...]) + ceb1[...], 0.0)
    zc = _mm(_bf(h), cew2[...]) + ceb2[...]
    z = jnp.concatenate([zs, zc], axis=1)

    zq0, i0 = _quant(z, cbs_t[...], cbs[...])
    s0 = jnp.sum((zq0 - z) ** 2)
    r = z - zq0
    rs_, rc_ = r[:, :H], r[:, H:]
    qs, qc = zq0[:, :H], zq0[:, H:]

    zq1, i1 = _quant(rs_, cbm0_t[...], cbm0[...])
    s1 = jnp.sum((zq1 - rs_) ** 2)
    rs2 = rs_ - zq1
    qs = qs + zq1
    zq2, i2 = _quant(rs2, cbm1_t[...], cbm1[...])
    s2 = jnp.sum((zq2 - rs2) ** 2)
    qs = qs + zq2

    zq3, i3 = _quant(rc_, cbc0_t[...], cbc0[...])
    s3 = jnp.sum((zq3 - rc_) ** 2)
    rc2 = rc_ - zq3
    qc = qc + zq3
    zq4, i4 = _quant(rc2, cbc1_t[...], cbc1[...])
    s4 = jnp.sum((zq4 - rc2) ** 2)
    qc = qc + zq4

    idx_ref[...] = jnp.concatenate([i0, i1, i2, i3, i4], axis=1)
    contrib = (1.25 / 5.0) * (s0 / (B * 256.0) + (s1 + s2 + s3 + s4) / (B * 128.0))
    loss_ref[...] = jnp.full((1, 1, 1), contrib, jnp.float32)

    h = jnp.maximum(_mm(_bf(qs), sdw0[...]) + sdb0[...], 0.0)
    h = jnp.maximum(_mm(_bf(h), sdw1[...]) + sdb1[...], 0.0)
    semo_ref[...] = _mm(_bf(h), sdw2[...]) + sdb2[...]
    h = jnp.maximum(_mm(_bf(qc), cdw0[...]) + cdb0[...], 0.0)
    h = jnp.maximum(_mm(_bf(h), cdw1[...]) + cdb1[...], 0.0)
    colo_ref[...] = _mm(_bf(h), cdw2[...]) + cdb2[...]


def _full_spec(shape):
    return pl.BlockSpec(shape, lambda i: (0,) * len(shape))


def kernel(x_semantic, x_collaborate,
           sem_enc_W0, sem_enc_b0, sem_enc_W1, sem_enc_b1, sem_enc_W2, sem_enc_b2,
           sem_dec_W0, sem_dec_b0, sem_dec_W1, sem_dec_b1, sem_dec_W2, sem_dec_b2,
           col_enc_W0, col_enc_b0, col_enc_W1, col_enc_b1, col_enc_W2, col_enc_b2,
           col_dec_W0, col_dec_b0, col_dec_W1, col_dec_b1, col_dec_W2, col_dec_b2,
           cb_shared_0, cb_sem_0, cb_sem_1, cb_col_0, cb_col_1):
    nt = B // TB
    b2 = lambda b: b.reshape(1, -1)
    bf = lambda w: w.astype(jnp.bfloat16)

    ins = [
        x_semantic, x_collaborate,
        bf(sem_enc_W0), b2(sem_enc_b0), bf(sem_enc_W1), b2(sem_enc_b1), bf(sem_enc_W2), b2(sem_enc_b2),
        bf(col_enc_W0), b2(col_enc_b0), bf(col_enc_W1), b2(col_enc_b1), bf(col_enc_W2), b2(col_enc_b2),
        bf(sem_dec_W0), b2(sem_dec_b0), bf(sem_dec_W1), b2(sem_dec_b1), bf(sem_dec_W2), b2(sem_dec_b2),
        bf(col_dec_W0), b2(col_dec_b0), bf(col_dec_W1), b2(col_dec_b1), bf(col_dec_W2), b2(col_dec_b2),
        bf(cb_shared_0.T), bf(cb_sem_0.T), bf(cb_sem_1.T), bf(cb_col_0.T), bf(cb_col_1.T),
        cb_shared_0, cb_sem_0, cb_sem_1, cb_col_0, cb_col_1,
    ]
    in_specs = [
        pl.BlockSpec((TB, 768), lambda i: (i, 0)),
        pl.BlockSpec((TB, 768), lambda i: (i, 0)),
    ] + [_full_spec(a.shape) for a in ins[2:]]

    sem_out, col_out, indices, loss_parts = pl.pallas_call(
        _fused_kernel,
        grid=(nt,),
        in_specs=in_specs,
        out_specs=[
            pl.BlockSpec((TB, 768), lambda i: (i, 0)),
            pl.BlockSpec((TB, 768), lambda i: (i, 0)),
            pl.BlockSpec((TB, 5), lambda i: (i, 0)),
            pl.BlockSpec((1, 1, 1), lambda i: (i, 0, 0)),
        ],
        out_shape=[
            jax.ShapeDtypeStruct((B, 768), jnp.float32),
            jax.ShapeDtypeStruct((B, 768), jnp.float32),
            jax.ShapeDtypeStruct((B, 5), jnp.int32),
            jax.ShapeDtypeStruct((nt, 1, 1), jnp.float32),
        ],
        compiler_params=pltpu.CompilerParams(
            dimension_semantics=("parallel",),
        ),
    )(*ins)

    return sem_out, col_out, jnp.sum(loss_parts), indices


# speedup vs baseline: 1.1423x; 1.1423x over previous
import jax
import jax.numpy as jnp
from jax import lax
from jax.experimental import pallas as pl
from jax.experimental.pallas import tpu as pltpu

B = 4096
TB = 512
H = 128
HI = lax.Precision.HIGHEST


def _mm(a_bf, b_bf):
    return jnp.dot(a_bf, b_bf, precision=lax.Precision.DEFAULT,
                   preferred_element_type=jnp.float32)


def _bf(x):
    return x.astype(jnp.bfloat16)


def _quant(zres, cb_t_bf, cb):
    rs = jnp.sum(zres * zres, axis=1, keepdims=True)
    cs = jnp.sum(cb * cb, axis=1)[None, :]
    d = rs + cs - 2.0 * _mm(_bf(zres), cb_t_bf)
    m = jnp.min(d, axis=1, keepdims=True)
    ii = lax.broadcasted_iota(jnp.int32, d.shape, 1)
    idx = jnp.min(jnp.where(d == m, ii, d.shape[1]), axis=1, keepdims=True)
    oh = (ii == idx).astype(jnp.float32)
    zq = jnp.dot(oh, cb, precision=HI, preferred_element_type=jnp.float32)
    return zq, idx


def _enc_kernel(xs_ref, xc_ref,
                sew0, seb0, sew1, seb1, sew2, seb2,
                cew0, ceb0, cew1, ceb1, cew2, ceb2,
                cbs_t, cbm0_t, cbm1_t, cbc0_t, cbc1_t,
                cbs, cbm0, cbm1, cbc0, cbc1,
                semq_ref, colq_ref, idx_ref, loss_ref,
                sw0s, sw1s, sw2s, cw0s, cw1s, cw2s, zscr):
    i = pl.program_id(0)
    nt = pl.num_programs(0) - 1

    @pl.when(i == 0)
    def _pack():
        sw0s[...] = _bf(sew0[...])
        sw1s[...] = _bf(sew1[...])
        sw2s[...] = _bf(sew2[...])
        cw0s[...] = _bf(cew0[...])
        cw1s[...] = _bf(cew1[...])
        cw2s[...] = _bf(cew2[...])

    z = zscr[...]
    zq0, i0 = _quant(z, cbs_t[...], cbs[...])
    s0 = jnp.sum((zq0 - z) ** 2)
    r = z - zq0
    rs_, rc_ = r[:, :H], r[:, H:]
    qs, qc = zq0[:, :H], zq0[:, H:]

    zq1, i1 = _quant(rs_, cbm0_t[...], cbm0[...])
    s1 = jnp.sum((zq1 - rs_) ** 2)
    rs2 = rs_ - zq1
    qs = qs + zq1
    zq2, i2 = _quant(rs2, cbm1_t[...], cbm1[...])
    s2 = jnp.sum((zq2 - rs2) ** 2)
    qs = qs + zq2

    zq3, i3 = _quant(rc_, cbc0_t[...], cbc0[...])
    s3 = jnp.sum((zq3 - rc_) ** 2)
    rc2 = rc_ - zq3
    qc = qc + zq3
    zq4, i4 = _quant(rc2, cbc1_t[...], cbc1[...])
    s4 = jnp.sum((zq4 - rc2) ** 2)
    qc = qc + zq4

    semq_ref[...] = qs
    colq_ref[...] = qc
    idx_ref[...] = jnp.concatenate([i0, i1, i2, i3, i4], axis=1)
    contrib = (1.25 / 5.0) * (s0 / (B * 256.0) + (s1 + s2 + s3 + s4) / (B * 128.0))
    contrib = jnp.where(i > 0, contrib, 0.0)
    prev = jnp.where(i == 0, jnp.zeros_like(loss_ref[...]), loss_ref[...])
    loss_ref[...] = prev + jnp.full((1, 1), contrib, jnp.float32)

    h = jnp.maximum(_mm(_bf(xs_ref[...]), sw0s[...]) + seb0[...], 0.0)
    h = jnp.maximum(_mm(_bf(h), sw1s[...]) + seb1[...], 0.0)
    zs = _mm(_bf(h), sw2s[...]) + seb2[...]
    h2 = jnp.maximum(_mm(_bf(xc_ref[...]), cw0s[...]) + ceb0[...], 0.0)
    h2 = jnp.maximum(_mm(_bf(h2), cw1s[...]) + ceb1[...], 0.0)
    zc = _mm(_bf(h2), cw2s[...]) + ceb2[...]
    zscr[...] = jnp.concatenate([zs, zc], axis=1)


def _dec_kernel(sq_ref, cq_ref,
                sdw0, sdb0, sdw1, sdb1, sdw2, sdb2,
                cdw0, cdb0, cdw1, cdb1, cdw2, cdb2,
                semo_ref, colo_ref,
                sw0s, sw1s, sw2s, cw0s, cw1s, cw2s):
    i = pl.program_id(0)

    @pl.when(i == 0)
    def _pack():
        sw0s[...] = _bf(sdw0[...])
        sw1s[...] = _bf(sdw1[...])
        sw2s[...] = _bf(sdw2[...])
        cw0s[...] = _bf(cdw0[...])
        cw1s[...] = _bf(cdw1[...])
        cw2s[...] = _bf(cdw2[...])

    h = jnp.maximum(_mm(_bf(sq_ref[...]), sw0s[...]) + sdb0[...], 0.0)
    h = jnp.maximum(_mm(_bf(h), sw1s[...]) + sdb1[...], 0.0)
    semo_ref[...] = _mm(_bf(h), sw2s[...]) + sdb2[...]
    h = jnp.maximum(_mm(_bf(cq_ref[...]), cw0s[...]) + cdb0[...], 0.0)
    h = jnp.maximum(_mm(_bf(h), cw1s[...]) + cdb1[...], 0.0)
    colo_ref[...] = _mm(_bf(h), cw2s[...]) + cdb2[...]


def _full_spec(shape):
    return pl.BlockSpec(shape, lambda i: (0,) * len(shape))


def kernel(x_semantic, x_collaborate,
           sem_enc_W0, sem_enc_b0, sem_enc_W1, sem_enc_b1, sem_enc_W2, sem_enc_b2,
           sem_dec_W0, sem_dec_b0, sem_dec_W1, sem_dec_b1, sem_dec_W2, sem_dec_b2,
           col_enc_W0, col_enc_b0, col_enc_W1, col_enc_b1, col_enc_W2, col_enc_b2,
           col_dec_W0, col_dec_b0, col_dec_W1, col_dec_b1, col_dec_W2, col_dec_b2,
           cb_shared_0, cb_sem_0, cb_sem_1, cb_col_0, cb_col_1):
    grid = (B // TB,)
    b2 = lambda b: b.reshape(1, -1)
    bf = lambda w: w.astype(jnp.bfloat16)

    enc_ins = [
        sem_enc_W0, b2(sem_enc_b0), sem_enc_W1, b2(sem_enc_b1), sem_enc_W2, b2(sem_enc_b2),
        col_enc_W0, b2(col_enc_b0), col_enc_W1, b2(col_enc_b1), col_enc_W2, b2(col_enc_b2),
        bf(cb_shared_0.T), bf(cb_sem_0.T), bf(cb_sem_1.T), bf(cb_col_0.T), bf(cb_col_1.T),
        cb_shared_0, cb_sem_0, cb_sem_1, cb_col_0, cb_col_1,
    ]
    nt = B // TB
    semq, colq, indices, loss = pl.pallas_call(
        _enc_kernel,
        grid=(nt + 1,),
        in_specs=[
            pl.BlockSpec((TB, 768), lambda i: (jnp.minimum(i, nt - 1), 0)),
            pl.BlockSpec((TB, 768), lambda i: (jnp.minimum(i, nt - 1), 0)),
        ] + [_full_spec(a.shape) for a in enc_ins],
        out_specs=[
            pl.BlockSpec((TB, H), lambda i: (jnp.maximum(i - 1, 0), 0)),
            pl.BlockSpec((TB, H), lambda i: (jnp.maximum(i - 1, 0), 0)),
            pl.BlockSpec((TB, 5), lambda i: (jnp.maximum(i - 1, 0), 0)),
            pl.BlockSpec((1, 1), lambda i: (0, 0)),
        ],
        out_shape=[
            jax.ShapeDtypeStruct((B, H), jnp.float32),
            jax.ShapeDtypeStruct((B, H), jnp.float32),
            jax.ShapeDtypeStruct((B, 5), jnp.int32),
            jax.ShapeDtypeStruct((1, 1), jnp.float32),
        ],
        scratch_shapes=[
            pltpu.VMEM((768, 2048), jnp.bfloat16),
            pltpu.VMEM((2048, 1024), jnp.bfloat16),
            pltpu.VMEM((1024, 128), jnp.bfloat16),
            pltpu.VMEM((768, 2048), jnp.bfloat16),
            pltpu.VMEM((2048, 1024), jnp.bfloat16),
            pltpu.VMEM((1024, 128), jnp.bfloat16),
            pltpu.VMEM((TB, 256), jnp.float32),
        ],
        compiler_params=pltpu.CompilerParams(
            dimension_semantics=("arbitrary",),
        ),
    )(x_semantic, x_collaborate, *enc_ins)

    dec_ins = [
        sem_dec_W0, b2(sem_dec_b0), sem_dec_W1, b2(sem_dec_b1), sem_dec_W2, b2(sem_dec_b2),
        col_dec_W0, b2(col_dec_b0), col_dec_W1, b2(col_dec_b1), col_dec_W2, b2(col_dec_b2),
    ]
    sem_out, col_out = pl.pallas_call(
        _dec_kernel,
        grid=grid,
        in_specs=[
            pl.BlockSpec((TB, H), lambda i: (i, 0)),
            pl.BlockSpec((TB, H), lambda i: (i, 0)),
        ] + [_full_spec(a.shape) for a in dec_ins],
        out_specs=[
            pl.BlockSpec((TB, 768), lambda i: (i, 0)),
            pl.BlockSpec((TB, 768), lambda i: (i, 0)),
        ],
        out_shape=[
            jax.ShapeDtypeStruct((B, 768), jnp.float32),
            jax.ShapeDtypeStruct((B, 768), jnp.float32),
        ],
        scratch_shapes=[
            pltpu.VMEM((128, 1024), jnp.bfloat16),
            pltpu.VMEM((1024, 2048), jnp.bfloat16),
            pltpu.VMEM((2048, 768), jnp.bfloat16),
            pltpu.VMEM((128, 1024), jnp.bfloat16),
            pltpu.VMEM((1024, 2048), jnp.bfloat16),
            pltpu.VMEM((2048, 768), jnp.bfloat16),
        ],
        compiler_params=pltpu.CompilerParams(
            dimension_semantics=("arbitrary",),
        ),
    )(semq, colq, *dec_ins)

    return sem_out, col_out, loss.reshape(()), indices
